# baseline (speedup 1.0000x reference)
import functools

import jax
import jax.numpy as jnp
from jax import lax
from jax.experimental import pallas as pl
from jax.experimental.pallas import tpu as pltpu
from jax.experimental.pallas import tpu_sc as plsc

VOCAB = 1000000
N_NEW = 2000
BATCH = 4096
HIST = 200
D = 32
PADW = 128
SUB = PADW // D

NC, NS, L = 2, 16, 16
NW = NC * NS
BW_ = BATCH // NW
BG = BW_ // L
ETILES = D // 8


def _body(idx_hbm, tb_hbm, out_hbm,
          idx_v, idxg_v, raw_v, t_v, sem_g0, sem_g1, sem_o):
    cid = lax.axis_index("c")
    sid = lax.axis_index("s")
    wid = sid * NC + cid
    b0 = wid * BW_
    iota = lax.iota(jnp.int32, L)
    sem_g = (sem_g0, sem_g1)

    pltpu.sync_copy(idx_hbm.at[pl.ds(b0, BW_)], idx_v)

    def build_and_fire(h, par):
        hv = jnp.full((L,), 0, jnp.int32) + h
        for bg in range(BG):
            b16 = bg * L + iota
            vec = plsc.load_gather(idx_v, [b16, hv])
            idxg_v[par, pl.ds(bg * L, L)] = vec * SUB
        pltpu.async_copy(tb_hbm.at[idxg_v.at[par]], raw_v.at[par], sem_g[par])

    def process(h, par):
        pltpu.make_async_copy(tb_hbm.at[idxg_v.at[par]], raw_v.at[par],
                              sem_g[par]).wait()

        @pl.when(h >= 2)
        def _():
            for te in range(ETILES):
                pltpu.make_async_copy(t_v.at[par, pl.ds(te * 8, 8)],
                                      out_hbm.at[h, te, wid], sem_o).wait()

        for cc in range(D):
            ccv = jnp.full((L,), cc, jnp.int32)
            for bg in range(BG):
                b16 = bg * L + iota
                vals = plsc.load_gather(raw_v.at[par], [b16, ccv])
                t_v[par, cc, pl.ds(bg * L, L)] = vals

        for te in range(ETILES):
            pltpu.async_copy(t_v.at[par, pl.ds(te * 8, 8)],
                             out_hbm.at[h, te, wid], sem_o)

    build_and_fire(0, 0)

    def h_pair(hp, _):
        for k in (0, 1):
            h = hp * 2 + k

            @pl.when(h + 1 < HIST)
            def _():
                build_and_fire(h + 1, 1 - k)

            process(h, k)
        return 0

    lax.fori_loop(0, HIST // 2, h_pair, 0)

    for par in range(2):
        for te in range(ETILES):
            pltpu.make_async_copy(t_v.at[par, pl.ds(te * 8, 8)],
                                  out_hbm.at[0, te, wid], sem_o).wait()


_mesh = plsc.VectorSubcoreMesh(
    core_axis_name="c", subcore_axis_name="s", num_cores=NC, num_subcores=NS)

_emb = functools.partial(
    pl.kernel,
    out_type=jax.ShapeDtypeStruct((HIST, ETILES, NW, 8, PADW), jnp.float32),
    mesh=_mesh,
    scratch_types=[
        pltpu.VMEM((BW_, HIST), jnp.int32),
        pltpu.VMEM((2, BW_), jnp.int32),
        pltpu.VMEM((2, BW_, D), jnp.float32),
        pltpu.VMEM((2, D, PADW), jnp.float32),
        pltpu.SemaphoreType.DMA,
        pltpu.SemaphoreType.DMA,
        pltpu.SemaphoreType.DMA,
    ],
    compiler_params=pltpu.CompilerParams(
        use_tc_tiling_on_sc=False, needs_layout_passes=False),
)(_body)


def kernel(input, table_base, table_new):
    full = jnp.concatenate([table_base, table_new], axis=0)
    fullp = jnp.pad(full, ((0, 0), (0, PADW - D)))
    tb32 = fullp.reshape((VOCAB + N_NEW) * SUB, D)
    out5 = _emb(input.astype(jnp.int32), tb32)
    return out5.transpose(2, 4, 0, 1, 3).reshape(BATCH, HIST, D)


# speedup vs baseline: 1.0094x; 1.0094x over previous
import functools

import jax
import jax.numpy as jnp
from jax import lax
from jax.experimental import pallas as pl
from jax.experimental.pallas import tpu as pltpu
from jax.experimental.pallas import tpu_sc as plsc

VOCAB = 1000000
N_NEW = 2000
BATCH = 4096
HIST = 200
D = 32
PADW = 128
SUB = PADW // D

NC, NS, L = 2, 16, 16
NW = NC * NS
BW_ = BATCH // NW
BG = BW_ // L
CH = 8
N_CHUNKS = HIST // CH
ETILES = D // 8


def _body(idx_hbm, tb_hbm, out_hbm,
          idx_v, idxg_v, raw_v, t_v, sem_g0, sem_g1, sem_o):
    cid = lax.axis_index("c")
    sid = lax.axis_index("s")
    wid = sid * NC + cid
    b0 = wid * BW_
    iota = lax.iota(jnp.int32, L)
    sem_g = (sem_g0, sem_g1)

    pltpu.sync_copy(idx_hbm.at[pl.ds(b0, BW_)], idx_v)

    def build_and_fire(c, par):
        for hh in range(CH):
            hv = jnp.full((L,), 0, jnp.int32) + (c * CH + hh)
            for bg in range(BG):
                b16 = bg * L + iota
                vec = plsc.load_gather(idx_v, [b16, hv])
                idxg_v[par, hh, pl.ds(bg * L, L)] = vec * SUB
        for hh in range(CH):
            pltpu.async_copy(tb_hbm.at[idxg_v.at[par, hh]],
                             raw_v.at[par, hh], sem_g[par])

    def drain_gathers(par):
        for hh in range(CH):
            pltpu.make_async_copy(tb_hbm.at[idxg_v.at[par, hh]],
                                  raw_v.at[par, hh], sem_g[par]).wait()

    def process_chunk(c, par):
        drain_gathers(par)

        def one_h(hh, _):
            tp = hh % 2
            h = c * CH + hh

            @pl.when(h >= 2)
            def _():
                for te in range(ETILES):
                    pltpu.make_async_copy(t_v.at[tp, pl.ds(te * 8, 8)],
                                          out_hbm.at[h, te, wid],
                                          sem_o).wait()

            for cc in range(D):
                ccv = jnp.full((L,), cc, jnp.int32)
                for bg in range(BG):
                    b16 = bg * L + iota
                    vals = plsc.load_gather(raw_v.at[par, hh], [b16, ccv])
                    t_v[tp, cc, pl.ds(bg * L, L)] = vals

            for te in range(ETILES):
                pltpu.async_copy(t_v.at[tp, pl.ds(te * 8, 8)],
                                 out_hbm.at[h, te, wid], sem_o)
            return 0

        lax.fori_loop(0, CH, one_h, 0)

    build_and_fire(0, 0)

    def chunk_pair(c2, _):
        for k in (0, 1):
            c = c2 * 2 + k

            @pl.when(c < N_CHUNKS)
            def _():
                @pl.when(c + 1 < N_CHUNKS)
                def _():
                    build_and_fire(c + 1, 1 - k)

                process_chunk(c, k)
        return 0

    lax.fori_loop(0, (N_CHUNKS + 1) // 2, chunk_pair, 0)

    for par in range(2):
        for te in range(ETILES):
            pltpu.make_async_copy(t_v.at[par, pl.ds(te * 8, 8)],
                                  out_hbm.at[0, te, wid], sem_o).wait()


_mesh = plsc.VectorSubcoreMesh(
    core_axis_name="c", subcore_axis_name="s", num_cores=NC, num_subcores=NS)

_emb = functools.partial(
    pl.kernel,
    out_type=jax.ShapeDtypeStruct((HIST, ETILES, NW, 8, PADW), jnp.float32),
    mesh=_mesh,
    scratch_types=[
        pltpu.VMEM((BW_, HIST), jnp.int32),
        pltpu.VMEM((2, CH, BW_), jnp.int32),
        pltpu.VMEM((2, CH, BW_, D), jnp.float32),
        pltpu.VMEM((2, D, PADW), jnp.float32),
        pltpu.SemaphoreType.DMA,
        pltpu.SemaphoreType.DMA,
        pltpu.SemaphoreType.DMA,
    ],
    compiler_params=pltpu.CompilerParams(
        use_tc_tiling_on_sc=False, needs_layout_passes=False),
)(_body)


def kernel(input, table_base, table_new):
    full = jnp.concatenate([table_base, table_new], axis=0)
    fullp = jnp.pad(full, ((0, 0), (0, PADW - D)))
    tb32 = fullp.reshape((VOCAB + N_NEW) * SUB, D)
    out5 = _emb(input.astype(jnp.int32), tb32)
    return out5.transpose(2, 4, 0, 1, 3).reshape(BATCH, HIST, D)


# speedup vs baseline: 1.3319x; 1.3195x over previous
import functools

import jax
import jax.numpy as jnp
from jax import lax
from jax.experimental import pallas as pl
from jax.experimental.pallas import tpu as pltpu
from jax.experimental.pallas import tpu_sc as plsc

VOCAB = 1000000
N_NEW = 2000
BATCH = 4096
HIST = 200
D = 32

NC, NS, L = 2, 16, 16
NW = NC * NS

NBLK = VOCAB // 128
TAILW = VOCAB - NBLK * 128
TN_BLK = N_NEW // 128
TN_TAILW = N_NEW - TN_BLK * 128
R_FULL = NBLK * 32
TAIL_ROWS = 520
ROWS_PAD = R_FULL + TAIL_ROWS
PER_W = (NBLK + NW - 1) // NW

_mesh = plsc.VectorSubcoreMesh(
    core_axis_name="c", subcore_axis_name="s", num_cores=NC, num_subcores=NS)


def _t_body(tbT_hbm, tail4_hbm, tb4_hbm, in_v, out_v, sem_i0, sem_i1, sem_o):
    cid = lax.axis_index("c")
    sid = lax.axis_index("s")
    wid = sid * NC + cid
    iota = lax.iota(jnp.int32, L)
    sem_i = (sem_i0, sem_i1)

    def fire_in(blk, par, width):
        pltpu.async_copy(tbT_hbm.at[pl.ds(0, 32), pl.ds(blk * 128, width)],
                         in_v.at[par, pl.ds(0, 32), pl.ds(0, width)],
                         sem_i[par])

    def wait_in(blk, par, width):
        pltpu.make_async_copy(
            tbT_hbm.at[pl.ds(0, 32), pl.ds(blk * 128, width)],
            in_v.at[par, pl.ds(0, 32), pl.ds(0, width)], sem_i[par]).wait()

    def transpose_block(par, width):
        @plsc.parallel_loop(0, width * 2, 1, unroll=8)
        def _(i):
            v = i // 2
            e0 = (i % 2) * L
            vals = plsc.load_gather(in_v.at[par], [e0 + iota,
                                                   jnp.full((L,), 0, jnp.int32) + v])
            out_v[par, (v * D + e0) // 128, pl.ds((v % 4) * D + e0, L)] = vals

    def out_dma(par, r0, nrows):
        return pltpu.async_copy(out_v.at[par, pl.ds(0, nrows)],
                                tb4_hbm.at[pl.ds(r0, nrows)], sem_o)

    def wait_out(par, r0, nrows):
        pltpu.make_async_copy(out_v.at[par, pl.ds(0, nrows)],
                              tb4_hbm.at[pl.ds(r0, nrows)], sem_o).wait()

    blk0 = wid
    fire_in(blk0, 0, 128)

    def pair_body(nb2, _):
        for k in (0, 1):
            nb = nb2 * 2 + k
            blk = wid + nb * NW

            @pl.when(blk < NBLK)
            def _():
                nblk = wid + (nb + 1) * NW

                @pl.when(nblk < NBLK)
                def _():
                    fire_in(nblk, 1 - k, 128)

                wait_in(blk, k, 128)

                @pl.when(nb >= 2)
                def _():
                    wait_out(k, blk * 32, 32)

                transpose_block(k, 128)
                out_dma(k, blk * 32, 32)
        return 0

    lax.fori_loop(0, (PER_W + 1) // 2, pair_body, 0)

    cnt = (NBLK - wid + NW - 1) // NW
    for k in (0, 1):
        ln = jnp.where(((cnt - 1) % 2) == k, cnt - 1, cnt - 2)

        @pl.when(ln >= 0)
        def _():
            wait_out(k, (wid + ln * NW) * 32, 32)

    stripe = TAIL_ROWS // 13
    for s in range(13):
        @pl.when(wid == 2 * s)
        def _():
            pltpu.sync_copy(
                tail4_hbm.at[pl.ds(s * stripe, stripe)],
                tb4_hbm.at[pl.ds(R_FULL + s * stripe, stripe)])


_relayout = functools.partial(
    pl.kernel,
    out_type=jax.ShapeDtypeStruct((ROWS_PAD, 128), jnp.float32),
    mesh=_mesh,
    scratch_types=[
        pltpu.VMEM((2, 32, 128), jnp.float32),
        pltpu.VMEM((2, 32, 128), jnp.float32),
        pltpu.SemaphoreType.DMA,
        pltpu.SemaphoreType.DMA,
        pltpu.SemaphoreType.DMA,
    ],
    compiler_params=pltpu.CompilerParams(
        use_tc_tiling_on_sc=True, needs_layout_passes=False),
)(_t_body)


BW_ = BATCH // NW
BG = BW_ // L
CH = 8
N_CHUNKS = HIST // CH
ETILES = D // 8


def _g_body(idx_hbm, tb_hbm, out_hbm,
            idx_v, idxg_v, raw_v, t_v, sem_g0, sem_g1, sem_o):
    cid = lax.axis_index("c")
    sid = lax.axis_index("s")
    wid = sid * NC + cid
    b0 = wid * BW_
    iota = lax.iota(jnp.int32, L)
    sem_g = (sem_g0, sem_g1)

    pltpu.sync_copy(idx_hbm.at[pl.ds(b0, BW_)], idx_v)

    def build_and_fire(c, par):
        @plsc.parallel_loop(0, CH * BG, 1, unroll=8)
        def _(i):
            hh = i // BG
            bg = i % BG
            hv = jnp.full((L,), 0, jnp.int32) + (c * CH + hh)
            b16 = bg * L + iota
            vec = plsc.load_gather(idx_v, [b16, hv])
            idxg_v[par, hh, pl.ds(bg * L, L)] = vec

        for hh in range(CH):
            pltpu.async_copy(tb_hbm.at[idxg_v.at[par, hh]],
                             raw_v.at[par, hh], sem_g[par])

    def process_chunk(c, par):
        for hh in range(CH):
            pltpu.make_async_copy(tb_hbm.at[idxg_v.at[par, hh]],
                                  raw_v.at[par, hh], sem_g[par]).wait()

        def one_h(hh, _):
            tp = hh % 2
            h = c * CH + hh

            @pl.when(h >= 2)
            def _():
                for te in range(ETILES):
                    pltpu.make_async_copy(t_v.at[tp, pl.ds(te * 8, 8)],
                                          out_hbm.at[h, te, wid],
                                          sem_o).wait()

            @plsc.parallel_loop(0, D * BG, 1, unroll=8)
            def _(i):
                cc = i // BG
                bg = i % BG
                ccv = jnp.full((L,), 0, jnp.int32) + cc
                b16 = bg * L + iota
                vals = plsc.load_gather(raw_v.at[par, hh], [b16, ccv])
                t_v[tp, cc, pl.ds(bg * L, L)] = vals

            for te in range(ETILES):
                pltpu.async_copy(t_v.at[tp, pl.ds(te * 8, 8)],
                                 out_hbm.at[h, te, wid], sem_o)
            return 0

        lax.fori_loop(0, CH, one_h, 0)

    build_and_fire(0, 0)

    def chunk_pair(c2, _):
        for k in (0, 1):
            c = c2 * 2 + k

            @pl.when(c < N_CHUNKS)
            def _():
                @pl.when(c + 1 < N_CHUNKS)
                def _():
                    build_and_fire(c + 1, 1 - k)

                process_chunk(c, k)
        return 0

    lax.fori_loop(0, (N_CHUNKS + 1) // 2, chunk_pair, 0)

    for par in range(2):
        for te in range(ETILES):
            pltpu.make_async_copy(t_v.at[par, pl.ds(te * 8, 8)],
                                  out_hbm.at[0, te, wid], sem_o).wait()


_gather = functools.partial(
    pl.kernel,
    out_type=jax.ShapeDtypeStruct((HIST, ETILES, NW, 8, 128), jnp.float32),
    mesh=_mesh,
    scratch_types=[
        pltpu.VMEM((BW_, HIST), jnp.int32),
        pltpu.VMEM((2, CH, BW_), jnp.int32),
        pltpu.VMEM((2, CH, BW_, D), jnp.float32),
        pltpu.VMEM((2, D, 128), jnp.float32),
        pltpu.SemaphoreType.DMA,
        pltpu.SemaphoreType.DMA,
        pltpu.SemaphoreType.DMA,
    ],
    compiler_params=pltpu.CompilerParams(
        use_tc_tiling_on_sc=False, needs_layout_passes=False),
)(_g_body)


def kernel(input, table_base, table_new):
    tail = jnp.concatenate([table_base[NBLK * 128:], table_new], axis=0)
    tail4 = jnp.pad(tail, ((0, TAIL_ROWS * 4 - TAILW - N_NEW), (0, 0)))
    tail4 = tail4.reshape(TAIL_ROWS, 128)
    tb4 = _relayout(table_base.T, tail4)
    tb32 = tb4.reshape(ROWS_PAD * 4, D)
    out5 = _gather(input.astype(jnp.int32), tb32)
    return out5.transpose(2, 4, 0, 1, 3).reshape(BATCH, HIST, D)
